# baseline (speedup 1.0000x reference)
import jax
import jax.numpy as jnp
from jax.experimental import pallas as pl
from jax.experimental.pallas import tpu as pltpu

_BB = 8


def _emb_kernel(idx_ref, emb_ref, wtop_ref, bias_ref, out_ref, rows_ref):
    nrows = rows_ref.shape[0]
    base = pl.program_id(0) * nrows
    for k in range(nrows):
        r = idx_ref[base + k]
        rows_ref[k : k + 1, :] = emb_ref[pl.ds(r, 1), :]
    out_ref[...] = (
        jnp.dot(rows_ref[...], wtop_ref[...], preferred_element_type=jnp.float32)
        + bias_ref[...]
    )


_PP = 208


def _enc_kernel(enc_ref, eaw_ref, eab_ref, hlw_ref, hlb_ref, clw_ref, clb_ref,
                encatt_ref, encbf_ref, h0_ref, c0_ref):
    bb, p, e = enc_ref.shape
    d = eaw_ref.shape[1]
    scale = jnp.full((1, p), 1.0 / p, jnp.float32)
    avgs = []
    for b in range(bb):
        enc_b = enc_ref[b]
        proj = (
            jnp.dot(enc_b, eaw_ref[...], preferred_element_type=jnp.float32)
            + eab_ref[...]
        ).astype(jnp.bfloat16)
        encatt_ref[b * _PP : b * _PP + p, :] = proj
        encatt_ref[b * _PP + p : (b + 1) * _PP, :] = jnp.zeros((_PP - p, d), jnp.bfloat16)
        encbf_ref[b * _PP : b * _PP + p, :] = enc_b.astype(jnp.bfloat16)
        encbf_ref[b * _PP + p : (b + 1) * _PP, :] = jnp.zeros((_PP - p, e), jnp.bfloat16)
        avgs.append(jnp.dot(scale, enc_b, preferred_element_type=jnp.float32))
    avg = jnp.concatenate(avgs, axis=0)
    h0_ref[...] = jnp.dot(avg, hlw_ref[...], preferred_element_type=jnp.float32) + hlb_ref[...]
    c0_ref[...] = jnp.dot(avg, clw_ref[...], preferred_element_type=jnp.float32) + clb_ref[...]


def _decode_kernel(enc_ref, encatt_ref, embp_ref, h0_ref, c0_ref, mask_ref,
                   daw_ref, dab_ref, atw_ref, atb_ref, fbw_ref, fbb_ref,
                   wawe_ref, whh_ref, h_out_ref, alpha_out_ref):
    t_steps, bb, _ = embp_ref.shape
    d = h0_ref.shape[1]
    p_real = alpha_out_ref.shape[2]

    lane = jax.lax.broadcasted_iota(jnp.int32, (bb, bb * _PP), 1)
    row = jax.lax.broadcasted_iota(jnp.int32, (bb, bb * _PP), 0)
    diag_mask = (lane // _PP) == row
    pad_lane = jax.lax.broadcasted_iota(jnp.int32, (bb, _PP), 1) < p_real

    def step(t, carry):
        h, c = carry
        h_bf = h.astype(jnp.bfloat16)
        dec = jnp.dot(h_bf, daw_ref[...], preferred_element_type=jnp.float32) + dab_ref[...]
        dec_bf = dec.astype(jnp.bfloat16)
        e_rows = []
        for b in range(bb):
            r = jnp.maximum(encatt_ref[b * _PP : (b + 1) * _PP, :] + dec_bf[b : b + 1, :], 0)
            e_rows.append(
                jax.lax.dot_general(
                    atw_ref[...], r,
                    dimension_numbers=(((1,), (1,)), ((), ())),
                    preferred_element_type=jnp.float32,
                )
            )
        e = jnp.concatenate(e_rows, axis=0) + atb_ref[...]
        e = jnp.where(pad_lane, e, -1e30)
        m = jnp.max(e, axis=1, keepdims=True)
        ex = jnp.exp(e - m)
        alpha = ex / jnp.sum(ex, axis=1, keepdims=True)
        alpha_tiled = jnp.tile(alpha, (1, bb))
        alpha_diag = jnp.where(diag_mask, alpha_tiled, 0.0).astype(jnp.bfloat16)
        awe = jnp.dot(alpha_diag, enc_ref[...], preferred_element_type=jnp.float32)
        gate = jax.nn.sigmoid(
            jnp.dot(h_bf, fbw_ref[...], preferred_element_type=jnp.float32) + fbb_ref[...]
        )
        awe_bf = (gate * awe).astype(jnp.bfloat16)
        embp_t = embp_ref[pl.ds(t, 1)].reshape(bb, -1)
        gates = (
            embp_t
            + jnp.dot(awe_bf, wawe_ref[...], preferred_element_type=jnp.float32)
            + jnp.dot(h_bf, whh_ref[...], preferred_element_type=jnp.float32)
        )
        i_g = gates[:, :d]
        f_g = gates[:, d : 2 * d]
        g_g = gates[:, 2 * d : 3 * d]
        o_g = gates[:, 3 * d :]
        c_new = jax.nn.sigmoid(f_g) * c + jax.nn.sigmoid(i_g) * jnp.tanh(g_g)
        h_new = jax.nn.sigmoid(o_g) * jnp.tanh(c_new)
        m_t = mask_ref[pl.ds(t, 1)].reshape(bb, 1)
        h_out_ref[pl.ds(t, 1)] = h_new[None]
        alpha_out_ref[pl.ds(t, 1)] = (alpha[:, :p_real] * m_t)[None]
        return h_new, c_new

    h0 = h0_ref[...]
    c0 = c0_ref[...]
    jax.lax.fori_loop(0, t_steps, step, (h0, c0))


def _fc_kernel(h_ref, w_ref, b_ref, m_ref, out_ref):
    acc = jnp.dot(h_ref[...], w_ref[...], preferred_element_type=jnp.float32)
    out_ref[...] = (acc + b_ref[...]) * m_ref[...]


def _cparams(sem):
    return pltpu.CompilerParams(
        dimension_semantics=sem, vmem_limit_bytes=64 * 1024 * 1024
    )


def kernel(encoder_out, encoded_captions, caption_lengths, emb_W,
           enc_att_W, enc_att_b, dec_att_W, dec_att_b, att_W, att_b,
           h_lin_W, h_lin_b, c_lin_W, c_lin_b, f_beta_W, f_beta_b,
           W_ih, W_hh, b_ih, b_hh, fc_W, fc_b):
    f32 = jnp.float32
    B, P, E = encoder_out.shape
    V, D = emb_W.shape
    L = encoded_captions.shape[1]
    T = L - 1
    G = 4 * D
    dec_len = caption_lengths - 1

    idx = encoded_captions[:, :T].T.reshape(-1).astype(jnp.int32)
    n_emb_blk = 8
    rows_per = (T * B) // n_emb_blk
    emb_part = pl.pallas_call(
        _emb_kernel,
        grid_spec=pltpu.PrefetchScalarGridSpec(
            num_scalar_prefetch=1,
            grid=(n_emb_blk,),
            in_specs=[
                pl.BlockSpec((V, D), lambda i, *_: (0, 0)),
                pl.BlockSpec((D, G), lambda i, *_: (0, 0)),
                pl.BlockSpec((1, G), lambda i, *_: (0, 0)),
            ],
            out_specs=pl.BlockSpec((rows_per, G), lambda i, *_: (i, 0)),
            scratch_shapes=[pltpu.VMEM((rows_per, D), f32)],
        ),
        out_shape=jax.ShapeDtypeStruct((T * B, G), f32),
        compiler_params=_cparams(("parallel",)),
    )(idx, emb_W, W_ih[:D], (b_ih + b_hh).reshape(1, G))
    embp3 = emb_part.reshape(T, B, G)

    bf16 = jnp.bfloat16
    nb = B // _BB
    encatt, encbf, h0, c0 = pl.pallas_call(
        _enc_kernel,
        grid=(nb,),
        in_specs=[
            pl.BlockSpec((_BB, P, E), lambda i: (i, 0, 0)),
            pl.BlockSpec((E, D), lambda i: (0, 0)),
            pl.BlockSpec((1, D), lambda i: (0, 0)),
            pl.BlockSpec((E, D), lambda i: (0, 0)),
            pl.BlockSpec((1, D), lambda i: (0, 0)),
            pl.BlockSpec((E, D), lambda i: (0, 0)),
            pl.BlockSpec((1, D), lambda i: (0, 0)),
        ],
        out_specs=[
            pl.BlockSpec((_BB * _PP, D), lambda i: (i, 0)),
            pl.BlockSpec((_BB * _PP, E), lambda i: (i, 0)),
            pl.BlockSpec((_BB, D), lambda i: (i, 0)),
            pl.BlockSpec((_BB, D), lambda i: (i, 0)),
        ],
        out_shape=[
            jax.ShapeDtypeStruct((B * _PP, D), bf16),
            jax.ShapeDtypeStruct((B * _PP, E), bf16),
            jax.ShapeDtypeStruct((B, D), f32),
            jax.ShapeDtypeStruct((B, D), f32),
        ],
        compiler_params=_cparams(("parallel",)),
    )(encoder_out, enc_att_W, enc_att_b.reshape(1, D),
      h_lin_W, h_lin_b.reshape(1, D), c_lin_W, c_lin_b.reshape(1, D))

    bd = 2 * _BB
    nbd = B // bd
    mask3 = (jnp.arange(T)[:, None] < dec_len[None, :]).astype(f32)[:, :, None]
    h_all, alphas_t = pl.pallas_call(
        _decode_kernel,
        grid=(nbd,),
        in_specs=[
            pl.BlockSpec((bd * _PP, E), lambda i: (i, 0)),
            pl.BlockSpec((bd * _PP, D), lambda i: (i, 0)),
            pl.BlockSpec((T, bd, G), lambda i: (0, i, 0)),
            pl.BlockSpec((bd, D), lambda i: (i, 0)),
            pl.BlockSpec((bd, D), lambda i: (i, 0)),
            pl.BlockSpec((T, bd, 1), lambda i: (0, i, 0)),
            pl.BlockSpec((D, D), lambda i: (0, 0)),
            pl.BlockSpec((1, D), lambda i: (0, 0)),
            pl.BlockSpec((1, D), lambda i: (0, 0)),
            pl.BlockSpec((1, 1), lambda i: (0, 0)),
            pl.BlockSpec((D, E), lambda i: (0, 0)),
            pl.BlockSpec((1, E), lambda i: (0, 0)),
            pl.BlockSpec((E, G), lambda i: (0, 0)),
            pl.BlockSpec((D, G), lambda i: (0, 0)),
        ],
        out_specs=[
            pl.BlockSpec((T, bd, D), lambda i: (0, i, 0)),
            pl.BlockSpec((T, bd, P), lambda i: (0, i, 0)),
        ],
        out_shape=[
            jax.ShapeDtypeStruct((T, B, D), f32),
            jax.ShapeDtypeStruct((T, B, P), f32),
        ],
        compiler_params=_cparams(("parallel",)),
    )(encbf, encatt, embp3, h0, c0, mask3,
      dec_att_W.astype(bf16), dec_att_b.reshape(1, D), att_W.T.astype(bf16),
      att_b.reshape(1, 1), f_beta_W.astype(bf16), f_beta_b.reshape(1, E),
      W_ih[D:].astype(bf16), W_hh.astype(bf16))

    mb = 336
    nbv = 1280
    grid_m = (T * B) // mb
    grid_v = (V + nbv - 1) // nbv
    preds = pl.pallas_call(
        _fc_kernel,
        grid=(grid_v, grid_m),
        in_specs=[
            pl.BlockSpec((mb, D), lambda j, i: (i, 0)),
            pl.BlockSpec((D, nbv), lambda j, i: (0, j)),
            pl.BlockSpec((1, nbv), lambda j, i: (0, j)),
            pl.BlockSpec((mb, 1), lambda j, i: (i, 0)),
        ],
        out_specs=pl.BlockSpec((mb, nbv), lambda j, i: (i, j)),
        out_shape=jax.ShapeDtypeStruct((T * B, V), f32),
        compiler_params=_cparams(("parallel", "arbitrary")),
    )(h_all.reshape(T * B, D), fc_W, fc_b.reshape(1, V),
      mask3.reshape(T * B, 1))

    predictions = preds.reshape(T, B, V).transpose(1, 0, 2)
    alphas = alphas_t.transpose(1, 0, 2)
    return predictions, encoded_captions, dec_len, alphas


# speedup vs baseline: 1.0112x; 1.0112x over previous
import jax
import jax.numpy as jnp
from jax.experimental import pallas as pl
from jax.experimental.pallas import tpu as pltpu

_BB = 8


def _emb_kernel(idx_ref, emb_ref, wtop_ref, bias_ref, out_ref, rows_ref):
    nrows = rows_ref.shape[0]
    base = pl.program_id(0) * nrows
    for k in range(nrows):
        r = idx_ref[base + k]
        rows_ref[k : k + 1, :] = emb_ref[pl.ds(r, 1), :]
    out_ref[...] = (
        jnp.dot(rows_ref[...], wtop_ref[...], preferred_element_type=jnp.float32)
        + bias_ref[...]
    )


_PP = 208


def _enc_kernel(enc_ref, eaw_ref, eab_ref, hlw_ref, hlb_ref, clw_ref, clb_ref,
                encatt_ref, encbf_ref, h0_ref, c0_ref):
    bb, p, e = enc_ref.shape
    d = eaw_ref.shape[1]
    scale = jnp.full((1, p), 1.0 / p, jnp.float32)
    avgs = []
    for b in range(bb):
        enc_b = enc_ref[b]
        proj = (
            jnp.dot(enc_b, eaw_ref[...], preferred_element_type=jnp.float32)
            + eab_ref[...]
        ).astype(jnp.bfloat16)
        encatt_ref[b * _PP : b * _PP + p, :] = proj
        encatt_ref[b * _PP + p : (b + 1) * _PP, :] = jnp.zeros((_PP - p, d), jnp.bfloat16)
        encbf_ref[b * _PP : b * _PP + p, :] = enc_b.astype(jnp.bfloat16)
        encbf_ref[b * _PP + p : (b + 1) * _PP, :] = jnp.zeros((_PP - p, e), jnp.bfloat16)
        avgs.append(jnp.dot(scale, enc_b, preferred_element_type=jnp.float32))
    avg = jnp.concatenate(avgs, axis=0)
    h0_ref[...] = jnp.dot(avg, hlw_ref[...], preferred_element_type=jnp.float32) + hlb_ref[...]
    c0_ref[...] = jnp.dot(avg, clw_ref[...], preferred_element_type=jnp.float32) + clb_ref[...]


def _decode_kernel(enc_ref, encatt_ref, embp_ref, h0_ref, c0_ref, mask_ref,
                   wh_ref, dab_ref, atw_ref, atb_ref, fbb_ref,
                   wawe_ref, h_out_ref, alpha_out_ref):
    t_steps, bb, _ = embp_ref.shape
    d = h0_ref.shape[1]
    g4 = embp_ref.shape[2]
    p_real = alpha_out_ref.shape[2]

    lane = jax.lax.broadcasted_iota(jnp.int32, (bb, bb * _PP), 1)
    row = jax.lax.broadcasted_iota(jnp.int32, (bb, bb * _PP), 0)
    diag_mask = (lane // _PP) == row
    pad_lane = jax.lax.broadcasted_iota(jnp.int32, (bb, _PP), 1) < p_real

    def step(t, carry):
        h, c = carry
        h_bf = h.astype(jnp.bfloat16)
        hp = jnp.dot(h_bf, wh_ref[...], preferred_element_type=jnp.float32)
        e_dim = fbb_ref.shape[1]
        dec = hp[:, :d] + dab_ref[...]
        gate = jax.nn.sigmoid(hp[:, d : d + e_dim] + fbb_ref[...])
        ghh = hp[:, d + e_dim :]
        dec_bf = dec.astype(jnp.bfloat16)
        e_rows = []
        for b in range(bb):
            r = jnp.maximum(encatt_ref[b * _PP : (b + 1) * _PP, :] + dec_bf[b : b + 1, :], 0)
            e_rows.append(
                jax.lax.dot_general(
                    atw_ref[...], r,
                    dimension_numbers=(((1,), (1,)), ((), ())),
                    preferred_element_type=jnp.float32,
                )
            )
        e = jnp.concatenate(e_rows, axis=0) + atb_ref[...]
        e = jnp.where(pad_lane, e, -1e30)
        m = jnp.max(e, axis=1, keepdims=True)
        ex = jnp.exp(e - m)
        alpha = ex / jnp.sum(ex, axis=1, keepdims=True)
        alpha_tiled = jnp.tile(alpha, (1, bb))
        alpha_diag = jnp.where(diag_mask, alpha_tiled, 0.0).astype(jnp.bfloat16)
        awe = jnp.dot(alpha_diag, enc_ref[...], preferred_element_type=jnp.float32)
        awe_bf = (gate * awe).astype(jnp.bfloat16)
        embp_t = embp_ref[pl.ds(t, 1)].reshape(bb, -1)
        gates = (
            embp_t
            + ghh
            + jnp.dot(awe_bf, wawe_ref[...], preferred_element_type=jnp.float32)
        )
        i_g = gates[:, :d]
        f_g = gates[:, d : 2 * d]
        g_g = gates[:, 2 * d : 3 * d]
        o_g = gates[:, 3 * d :]
        c_new = jax.nn.sigmoid(f_g) * c + jax.nn.sigmoid(i_g) * jnp.tanh(g_g)
        h_new = jax.nn.sigmoid(o_g) * jnp.tanh(c_new)
        m_t = mask_ref[pl.ds(t, 1)].reshape(bb, 1)
        h_out_ref[pl.ds(t, 1)] = h_new[None]
        alpha_out_ref[pl.ds(t, 1)] = (alpha[:, :p_real] * m_t)[None]
        return h_new, c_new

    h0 = h0_ref[...]
    c0 = c0_ref[...]
    jax.lax.fori_loop(0, t_steps, step, (h0, c0))


def _fc_kernel(h_ref, w_ref, b_ref, m_ref, out_ref):
    acc = jnp.dot(h_ref[...], w_ref[...], preferred_element_type=jnp.float32)
    out_ref[...] = (acc + b_ref[...]) * m_ref[...]


def _cparams(sem):
    return pltpu.CompilerParams(
        dimension_semantics=sem, vmem_limit_bytes=64 * 1024 * 1024
    )


def kernel(encoder_out, encoded_captions, caption_lengths, emb_W,
           enc_att_W, enc_att_b, dec_att_W, dec_att_b, att_W, att_b,
           h_lin_W, h_lin_b, c_lin_W, c_lin_b, f_beta_W, f_beta_b,
           W_ih, W_hh, b_ih, b_hh, fc_W, fc_b):
    f32 = jnp.float32
    B, P, E = encoder_out.shape
    V, D = emb_W.shape
    L = encoded_captions.shape[1]
    T = L - 1
    G = 4 * D
    dec_len = caption_lengths - 1

    idx = encoded_captions[:, :T].T.reshape(-1).astype(jnp.int32)
    n_emb_blk = 8
    rows_per = (T * B) // n_emb_blk
    emb_part = pl.pallas_call(
        _emb_kernel,
        grid_spec=pltpu.PrefetchScalarGridSpec(
            num_scalar_prefetch=1,
            grid=(n_emb_blk,),
            in_specs=[
                pl.BlockSpec((V, D), lambda i, *_: (0, 0)),
                pl.BlockSpec((D, G), lambda i, *_: (0, 0)),
                pl.BlockSpec((1, G), lambda i, *_: (0, 0)),
            ],
            out_specs=pl.BlockSpec((rows_per, G), lambda i, *_: (i, 0)),
            scratch_shapes=[pltpu.VMEM((rows_per, D), f32)],
        ),
        out_shape=jax.ShapeDtypeStruct((T * B, G), f32),
        compiler_params=_cparams(("parallel",)),
    )(idx, emb_W, W_ih[:D], (b_ih + b_hh).reshape(1, G))
    embp3 = emb_part.reshape(T, B, G)

    bf16 = jnp.bfloat16
    nb = B // _BB
    encatt, encbf, h0, c0 = pl.pallas_call(
        _enc_kernel,
        grid=(nb,),
        in_specs=[
            pl.BlockSpec((_BB, P, E), lambda i: (i, 0, 0)),
            pl.BlockSpec((E, D), lambda i: (0, 0)),
            pl.BlockSpec((1, D), lambda i: (0, 0)),
            pl.BlockSpec((E, D), lambda i: (0, 0)),
            pl.BlockSpec((1, D), lambda i: (0, 0)),
            pl.BlockSpec((E, D), lambda i: (0, 0)),
            pl.BlockSpec((1, D), lambda i: (0, 0)),
        ],
        out_specs=[
            pl.BlockSpec((_BB * _PP, D), lambda i: (i, 0)),
            pl.BlockSpec((_BB * _PP, E), lambda i: (i, 0)),
            pl.BlockSpec((_BB, D), lambda i: (i, 0)),
            pl.BlockSpec((_BB, D), lambda i: (i, 0)),
        ],
        out_shape=[
            jax.ShapeDtypeStruct((B * _PP, D), bf16),
            jax.ShapeDtypeStruct((B * _PP, E), bf16),
            jax.ShapeDtypeStruct((B, D), f32),
            jax.ShapeDtypeStruct((B, D), f32),
        ],
        compiler_params=_cparams(("parallel",)),
    )(encoder_out, enc_att_W, enc_att_b.reshape(1, D),
      h_lin_W, h_lin_b.reshape(1, D), c_lin_W, c_lin_b.reshape(1, D))

    bd = 2 * _BB
    nbd = B // bd
    mask3 = (jnp.arange(T)[:, None] < dec_len[None, :]).astype(f32)[:, :, None]
    h_all, alphas_t = pl.pallas_call(
        _decode_kernel,
        grid=(nbd,),
        in_specs=[
            pl.BlockSpec((bd * _PP, E), lambda i: (i, 0)),
            pl.BlockSpec((bd * _PP, D), lambda i: (i, 0)),
            pl.BlockSpec((T, bd, G), lambda i: (0, i, 0)),
            pl.BlockSpec((bd, D), lambda i: (i, 0)),
            pl.BlockSpec((bd, D), lambda i: (i, 0)),
            pl.BlockSpec((T, bd, 1), lambda i: (0, i, 0)),
            pl.BlockSpec((D, D + E + G), lambda i: (0, 0)),
            pl.BlockSpec((1, D), lambda i: (0, 0)),
            pl.BlockSpec((1, D), lambda i: (0, 0)),
            pl.BlockSpec((1, 1), lambda i: (0, 0)),
            pl.BlockSpec((1, E), lambda i: (0, 0)),
            pl.BlockSpec((E, G), lambda i: (0, 0)),
        ],
        out_specs=[
            pl.BlockSpec((T, bd, D), lambda i: (0, i, 0)),
            pl.BlockSpec((T, bd, P), lambda i: (0, i, 0)),
        ],
        out_shape=[
            jax.ShapeDtypeStruct((T, B, D), f32),
            jax.ShapeDtypeStruct((T, B, P), f32),
        ],
        compiler_params=_cparams(("parallel",)),
    )(encbf, encatt, embp3, h0, c0, mask3,
      jnp.concatenate([dec_att_W, f_beta_W, W_hh], axis=1).astype(bf16),
      dec_att_b.reshape(1, D), att_W.T.astype(bf16),
      att_b.reshape(1, 1), f_beta_b.reshape(1, E),
      W_ih[D:].astype(bf16))

    mb = 336
    nbv = 1280
    grid_m = (T * B) // mb
    grid_v = (V + nbv - 1) // nbv
    preds = pl.pallas_call(
        _fc_kernel,
        grid=(grid_v, grid_m),
        in_specs=[
            pl.BlockSpec((mb, D), lambda j, i: (i, 0)),
            pl.BlockSpec((D, nbv), lambda j, i: (0, j)),
            pl.BlockSpec((1, nbv), lambda j, i: (0, j)),
            pl.BlockSpec((mb, 1), lambda j, i: (i, 0)),
        ],
        out_specs=pl.BlockSpec((mb, nbv), lambda j, i: (i, j)),
        out_shape=jax.ShapeDtypeStruct((T * B, V), f32),
        compiler_params=_cparams(("parallel", "arbitrary")),
    )(h_all.reshape(T * B, D), fc_W, fc_b.reshape(1, V),
      mask3.reshape(T * B, 1))

    predictions = preds.reshape(T, B, V).transpose(1, 0, 2)
    alphas = alphas_t.transpose(1, 0, 2)
    return predictions, encoded_captions, dec_len, alphas
